# baseline (speedup 1.0000x reference)
import jax
import jax.numpy as jnp
from jax import lax
from jax.experimental import pallas as pl
from jax.experimental.pallas import tpu as pltpu
from jax.experimental.pallas import tpu_sc as plsc

_NUM_FIELDS = 26
_FIELD_DIM = 38462
_EMBED_DIM = 16
_BATCH = 16384
_TOTAL = _NUM_FIELDS * _FIELD_DIM

_NC, _NS = 2, 16
_NW = _NC * _NS
_B_PER_W = _BATCH // _NW
_CHUNK = 128
_C_PER_F = _B_PER_W // _CHUNK
_ROWS_W = _NUM_FIELDS * _C_PER_F


def _body(xt_hbm, bias_hbm, emb_hbm, fc_hbm, oemb_hbm, olr_hbm,
          xt_v, fbuf0, fbuf1, fbuf2, fcbuf, lrbuf, bias_v, gsem, fsem, wsem):
    wid = lax.axis_index("s") * _NC + lax.axis_index("c")
    bbase = wid * _B_PER_W
    crow = wid * _C_PER_F

    st_descs = [
        pltpu.async_copy(
            xt_hbm.at[pl.ds(f * (_BATCH // _CHUNK) + crow, _C_PER_F)],
            xt_v.at[pl.ds(f * _C_PER_F, _C_PER_F)], wsem)
        for f in range(_NUM_FIELDS)
    ]
    pltpu.sync_copy(bias_hbm, bias_v)
    for d in st_descs:
        d.wait()

    def _add_off(c, carry):
        for f in range(_NUM_FIELDS):
            for j in range(_CHUNK // 16):
                sl = pl.ds(j * 16, 16)
                r = f * _C_PER_F + c
                xt_v[r, sl] = xt_v[r, sl] + (f * _FIELD_DIM)
        return carry

    lax.fori_loop(0, _C_PER_F, _add_off, 0)

    bufs = (fbuf0, fbuf1, fbuf2)
    _NBUF = len(bufs)

    def _issue(f, buf):
        descs = []
        for c in range(_C_PER_F):
            descs.append(pltpu.async_copy(
                emb_hbm.at[xt_v.at[f * _C_PER_F + c]],
                buf.at[pl.ds(c * _CHUNK, _CHUNK)], gsem))
        return descs

    def _issue_fc(f):
        descs = []
        for c in range(_C_PER_F):
            descs.append(pltpu.async_copy(
                fc_hbm.at[xt_v.at[f * _C_PER_F + c]],
                fcbuf.at[pl.ds(f * _B_PER_W + c * _CHUNK, _CHUNK)], fsem))
        return descs

    fc_descs = []
    wb_descs = [None] * _NBUF
    g_descs = [None] * _NBUF
    for f in range(_NBUF - 1):
        g_descs[f % _NBUF] = _issue(f, bufs[f % _NBUF])
        fc_descs += _issue_fc(f)
    for f in range(_NUM_FIELDS):
        nf = f + _NBUF - 1
        if nf < _NUM_FIELDS:
            if wb_descs[nf % _NBUF] is not None:
                wb_descs[nf % _NBUF].wait()
                wb_descs[nf % _NBUF] = None
            g_descs[nf % _NBUF] = _issue(nf, bufs[nf % _NBUF])
            fc_descs += _issue_fc(nf)
        for d in g_descs[f % _NBUF]:
            d.wait()
        wb_descs[f % _NBUF] = pltpu.async_copy(
            bufs[f % _NBUF], oemb_hbm.at[f, pl.ds(bbase, _B_PER_W)], wsem)
    for d in wb_descs:
        if d is not None:
            d.wait()
    for d in fc_descs:
        d.wait()

    def _reduce(t, carry):
        acc = bias_v[...]
        for f in range(_NUM_FIELDS):
            acc = acc + fcbuf[pl.ds(f * _B_PER_W + t * 16, 16)]
        lrbuf[pl.ds(t * 16, 16)] = acc
        return carry

    lax.fori_loop(0, _B_PER_W // 16, _reduce, 0)
    pltpu.sync_copy(lrbuf, olr_hbm.at[pl.ds(bbase, _B_PER_W)])


def kernel(x, emb_table, fc_table, bias):
    xt2 = x.T.reshape(_BATCH * _NUM_FIELDS // _CHUNK, _CHUNK)
    bias16 = jnp.broadcast_to(bias, (16,))
    fc_flat = fc_table.reshape(_TOTAL)

    mesh = plsc.VectorSubcoreMesh(core_axis_name="c", subcore_axis_name="s",
                                  num_cores=_NC, num_subcores=_NS)
    run = pl.kernel(
        _body,
        out_type=[
            jax.ShapeDtypeStruct((_NUM_FIELDS, _BATCH, _EMBED_DIM),
                                 jnp.float32),
            jax.ShapeDtypeStruct((_BATCH,), jnp.float32),
        ],
        mesh=mesh,
        compiler_params=pltpu.CompilerParams(use_tc_tiling_on_sc=False),
        scratch_types=[
            pltpu.VMEM((_ROWS_W, _CHUNK), jnp.int32),
            pltpu.VMEM((_B_PER_W, _EMBED_DIM), jnp.float32),
            pltpu.VMEM((_B_PER_W, _EMBED_DIM), jnp.float32),
            pltpu.VMEM((_B_PER_W, _EMBED_DIM), jnp.float32),
            pltpu.VMEM((_NUM_FIELDS * _B_PER_W,), jnp.float32),
            pltpu.VMEM((_B_PER_W,), jnp.float32),
            pltpu.VMEM((16,), jnp.float32),
            pltpu.SemaphoreType.DMA,
            pltpu.SemaphoreType.DMA,
            pltpu.SemaphoreType.DMA,
        ],
    )
    oemb, olr = run(xt2, bias16, emb_table, fc_flat)
    return jnp.transpose(oemb, (1, 0, 2)), olr.reshape(_BATCH, 1)
